# baseline (speedup 1.0000x reference)
import functools

import jax
import jax.numpy as jnp
from jax import lax
from jax.experimental import pallas as pl
from jax.experimental.pallas import tpu as pltpu
from jax.experimental.pallas import tpu_sc as plsc

_B = 4096
_DIC = 100
_D = 64
_NE = 6
_NPOS = _B * _DIC
_DP = _DIC // 2
_NPAIR = _B * _DP
_NC = 2
_NS = 16
_NW = _NC * _NS
_PER_W = _NPAIR // _NW
_CH = 128
_NCH = _PER_W // _CH
_NB = 2
_BB = 128


def _tab_body(nm2_ref, vt_ref, out_ref):
    nm2 = nm2_ref[...]
    zero = jnp.zeros((_D,), jnp.float32)
    for e0 in range(_NE + 1):
        left = vt_ref[e0] if e0 < _NE else zero
        for e1 in range(_NE + 1):
            right = vt_ref[e1] if e1 < _NE else zero
            out_ref[e0, e1] = nm2 + jnp.concatenate([left, right], axis=-1)


def _build_table(name_embedding, value_table):
    out = pl.pallas_call(
        _tab_body,
        out_shape=jax.ShapeDtypeStruct((_NE + 1, _NE + 1, _DP, 2 * _D), jnp.float32),
    )(name_embedding.reshape(_DP, 2 * _D), value_table)
    return out.reshape((_NE + 1) * (_NE + 1) * _DP, 2 * _D)


_RPW = _B // _NW


def _sc_body(ctab_h, xc_h, ys_h, dpl_h, out_h,
             xc_v, ys_v, dp_v, i_v, stage_v, sem_g, sem_s):
    wid = lax.axis_index("s") * _NC + lax.axis_index("c")
    base0 = wid * _PER_W
    row0 = wid * _RPW
    pltpu.sync_copy(xc_h.at[pl.ds(base0, _PER_W)], xc_v)
    pltpu.sync_copy(ys_h.at[pl.ds(base0, _PER_W)], ys_v)
    pltpu.sync_copy(dpl_h, dp_v)

    def idx_row(r, carry):
        for off in (0, 16, 32, 34):
            sl = pl.ds(r * _DP + off, 16)
            xc = xc_v[sl]
            dp = dp_v[pl.ds(off, 16)]
            yv = ys_v[sl]
            xe = xc & 7
            xo = xc >> 3
            d0 = dp * 2
            e0 = jnp.where(d0 == yv, _NE, xe)
            e1 = jnp.where(d0 + 1 == yv, _NE, xo)
            i_v[r, pl.ds(off, 16)] = (e0 * (_NE + 1) + e1) * _DP + dp
        return carry

    lax.fori_loop(0, _RPW, idx_row, 0)

    def start_gather(r, b):
        pltpu.async_copy(ctab_h.at[i_v.at[r]], stage_v.at[b], sem_g)

    def wait_gather(r, b):
        pltpu.make_async_copy(ctab_h.at[i_v.at[r]], stage_v.at[b], sem_g).wait()

    for b in range(_NB):
        start_gather(b, b)

    def outer(t, carry):
        r0 = t * _NB
        for b in range(_NB):
            r = r0 + b
            wait_gather(r, b)
            pltpu.async_copy(stage_v.at[b], out_h.at[row0 + r], sem_s)
            pltpu.make_async_copy(
                stage_v.at[b], out_h.at[row0 + r], sem_s).wait()

            @pl.when(r + _NB < _RPW)
            def _():
                start_gather(r + _NB, b)
        return carry

    lax.fori_loop(0, _RPW // _NB, outer, 0)


def _sc_gather(ctab, xc, ys, dpl):
    mesh = plsc.VectorSubcoreMesh(core_axis_name="c", subcore_axis_name="s")
    run = functools.partial(
        pl.kernel,
        out_type=jax.ShapeDtypeStruct((_B, _DP, 2 * _D), jnp.float32),
        mesh=mesh,
        scratch_types=[
            pltpu.VMEM((_PER_W,), jnp.int32),
            pltpu.VMEM((_PER_W,), jnp.int32),
            pltpu.VMEM((_DP,), jnp.int32),
            pltpu.VMEM((_RPW, _DP), jnp.int32),
            pltpu.VMEM((_NB, _DP, 2 * _D), jnp.float32),
            pltpu.SemaphoreType.DMA,
            pltpu.SemaphoreType.DMA,
        ],
    )(_sc_body)
    return run(ctab, xc, ys, dpl)


def _unpack_body(mid_ref, out_ref):
    v = mid_ref[...]
    out_ref[:, pl.Slice(0, _DP, 2), :] = v[:, :, 0:_D]
    out_ref[:, pl.Slice(1, _DP, 2), :] = v[:, :, _D:]


def _tc_unpack(mid):
    return pl.pallas_call(
        _unpack_body,
        grid=(_B // _BB,),
        in_specs=[pl.BlockSpec((_BB, _DP, 2 * _D), lambda i: (i, 0, 0))],
        out_specs=pl.BlockSpec((_BB, _DIC, _D), lambda i: (i, 0, 0)),
        out_shape=jax.ShapeDtypeStruct((_B, _DIC, _D), jnp.float32),
    )(mid)


@jax.jit
def kernel(x, y, name_embedding, value_table):
    x = x.astype(jnp.int32)
    y = y.astype(jnp.int32)
    ctab = _build_table(name_embedding, value_table)
    pair_w = jnp.array([1, 8], dtype=jnp.int32)
    xc = jnp.sum(x.reshape(_NPAIR, 2) * pair_w[None, :], axis=1).reshape(_NPAIR)
    ys = jnp.repeat(y, _DP)
    dpl = jnp.arange(_DP, dtype=jnp.int32)
    mid = _sc_gather(ctab, xc, ys, dpl)
    return _tc_unpack(mid)
